# baseline (speedup 1.0000x reference)
import jax
import jax.numpy as jnp
from jax import lax
from jax.experimental import pallas as pl
from jax.experimental.pallas import tpu as pltpu
from jax.experimental.pallas import tpu_sc as plsc

D = 128
E = 320000
NC, NS, L = 2, 16, 16
NW = NC * NS
EW = E // NW
C = 80
K = EW // C
G = C // L
PS = L + 1
NSLOT = 4


def _body(z_hbm, src_hbm, dst_hbm, out_hbm,
          idx_s, idx_d, rowbufs, p1, out_buf, sems):
    wid = lax.axis_index("c") * NS + lax.axis_index("s")
    base = wid * EW
    iota = lax.iota(jnp.int32, L)
    iota_ps = iota * PS

    slots = tuple(
        (rowbufs[2 * s], rowbufs[2 * s + 1], sems[2 * s], sems[2 * s + 1])
        for s in range(NSLOT)
    )

    pltpu.sync_copy(src_hbm.at[pl.ds(base, EW)], idx_s)
    pltpu.sync_copy(dst_hbm.at[pl.ds(base, EW)], idx_d)

    def issue(c, slot):
        sr, dr, ss, sd = slots[slot]
        pltpu.async_copy(z_hbm.at[idx_s.at[pl.ds(c * C, C)]], sr, ss)
        pltpu.async_copy(z_hbm.at[idx_d.at[pl.ds(c * C, C)]], dr, sd)

    def wait(c, slot):
        sr, dr, ss, sd = slots[slot]
        pltpu.make_async_copy(z_hbm.at[idx_s.at[pl.ds(c * C, C)]], sr, ss).wait()
        pltpu.make_async_copy(z_hbm.at[idx_d.at[pl.ds(c * C, C)]], dr, sd).wait()

    def compute(c, slot):
        sr, dr, _, _ = slots[slot]

        def group_body(g, carry):
            for i in range(L):
                e = g * L + i
                prods = []
                for k in range(D // L):
                    vs = sr[e, pl.ds(k * L, L)]
                    vd = dr[e, pl.ds(k * L, L)]
                    prods.append(vs * vd)
                while len(prods) > 1:
                    prods = [a + b for a, b in zip(prods[::2], prods[1::2])]
                p1[pl.ds(i * PS, L)] = prods[0]
            acc = plsc.load_gather(p1, [iota_ps])
            for j in range(1, L):
                acc = acc + plsc.load_gather(p1, [iota_ps + j])
            out = 1.0 / (1.0 + jnp.exp(-acc))
            out_buf[pl.ds(c * C + g * L, L)] = out
            return carry

        lax.fori_loop(0, G, group_body, 0, unroll=False)

    for s in range(NSLOT):
        issue(s, s)

    def quad_body(q, carry):
        for s in range(NSLOT):
            c = NSLOT * q + s
            wait(c, s)
            compute(c, s)

            @pl.when(c + NSLOT < K)
            def _():
                issue(c + NSLOT, s)

        return carry

    lax.fori_loop(0, K // NSLOT, quad_body, 0, unroll=False)
    c_tail = K // NSLOT * NSLOT
    wait(c_tail, 0)
    compute(c_tail, 0)

    pltpu.sync_copy(out_buf, out_hbm.at[pl.ds(base, EW)])


@jax.jit
def _gae_decode(z, src, dst):
    mesh = plsc.VectorSubcoreMesh(core_axis_name="c", subcore_axis_name="s")
    return pl.kernel(
        _body,
        out_type=jax.ShapeDtypeStruct((E,), jnp.float32),
        mesh=mesh,
        compiler_params=pltpu.CompilerParams(needs_layout_passes=False),
        scratch_types=[
            pltpu.VMEM((EW,), jnp.int32),
            pltpu.VMEM((EW,), jnp.int32),
            [pltpu.VMEM((C, D), jnp.float32) for _ in range(2 * NSLOT)],
            pltpu.VMEM((L * PS,), jnp.float32),
            pltpu.VMEM((EW,), jnp.float32),
            [pltpu.SemaphoreType.DMA for _ in range(2 * NSLOT)],
        ],
    )(z, src, dst)


def kernel(z, edge_index):
    src = edge_index[0].astype(jnp.int32)
    dst = edge_index[1].astype(jnp.int32)
    return _gae_decode(z, src, dst)


# speedup vs baseline: 1.0726x; 1.0726x over previous
import jax
import jax.numpy as jnp
from jax import lax
from jax.experimental import pallas as pl
from jax.experimental.pallas import tpu as pltpu
from jax.experimental.pallas import tpu_sc as plsc

D = 128
E = 320000
NC, NS, L = 2, 16, 16
NW = NC * NS
EW = E // NW
C = 80
K = EW // C
G = C // L
PS = L + 1
NSLOT = 2


def _body(z_hbm, src_hbm, dst_hbm, out_hbm,
          idx_s, idx_d, rowbufs, p1, out_buf, sems):
    wid = lax.axis_index("c") * NS + lax.axis_index("s")
    base = wid * EW
    iota = lax.iota(jnp.int32, L)
    iota_ps = iota * PS

    slots = tuple(
        (rowbufs[2 * s], rowbufs[2 * s + 1], sems[2 * s], sems[2 * s + 1])
        for s in range(NSLOT)
    )

    pltpu.sync_copy(src_hbm.at[pl.ds(base, EW)], idx_s)
    pltpu.sync_copy(dst_hbm.at[pl.ds(base, EW)], idx_d)

    def issue(c, slot):
        sr, dr, ss, sd = slots[slot]
        pltpu.async_copy(z_hbm.at[idx_s.at[pl.ds(c * C, C)]], sr, ss)
        pltpu.async_copy(z_hbm.at[idx_d.at[pl.ds(c * C, C)]], dr, sd)

    def wait(c, slot):
        sr, dr, ss, sd = slots[slot]
        pltpu.make_async_copy(z_hbm.at[idx_s.at[pl.ds(c * C, C)]], sr, ss).wait()
        pltpu.make_async_copy(z_hbm.at[idx_d.at[pl.ds(c * C, C)]], dr, sd).wait()

    def compute(c, slot):
        sr, dr, _, _ = slots[slot]

        def group_body(g, carry):
            for i in range(L):
                e = g * L + i
                prods = []
                for k in range(D // L):
                    vs = sr[e, pl.ds(k * L, L)]
                    vd = dr[e, pl.ds(k * L, L)]
                    prods.append(vs * vd)
                while len(prods) > 1:
                    prods = [a + b for a, b in zip(prods[::2], prods[1::2])]
                p1[pl.ds(i * PS, L)] = prods[0]
            acc = plsc.load_gather(p1, [iota_ps])
            for j in range(1, L):
                acc = acc + plsc.load_gather(p1, [iota_ps + j])
            out = 1.0 / (1.0 + jnp.exp(-acc))
            out_buf[pl.ds(c * C + g * L, L)] = out
            return carry

        lax.fori_loop(0, G, group_body, 0, unroll=False)

    for s in range(NSLOT):
        issue(s, s)

    def quad_body(q, carry):
        for s in range(NSLOT):
            c = NSLOT * q + s
            wait(c, s)
            compute(c, s)

            @pl.when(c + NSLOT < K)
            def _():
                issue(c + NSLOT, s)

        return carry

    lax.fori_loop(0, K // NSLOT, quad_body, 0, unroll=False)
    c_tail = K // NSLOT * NSLOT
    wait(c_tail, 0)
    compute(c_tail, 0)

    pltpu.sync_copy(out_buf, out_hbm.at[pl.ds(base, EW)])


@jax.jit
def _gae_decode(z, src, dst):
    mesh = plsc.VectorSubcoreMesh(core_axis_name="c", subcore_axis_name="s")
    return pl.kernel(
        _body,
        out_type=jax.ShapeDtypeStruct((E,), jnp.float32),
        mesh=mesh,
        compiler_params=pltpu.CompilerParams(needs_layout_passes=False),
        scratch_types=[
            pltpu.VMEM((EW,), jnp.int32),
            pltpu.VMEM((EW,), jnp.int32),
            [pltpu.VMEM((C, D), jnp.float32) for _ in range(2 * NSLOT)],
            pltpu.VMEM((L * PS,), jnp.float32),
            pltpu.VMEM((EW,), jnp.float32),
            [pltpu.SemaphoreType.DMA for _ in range(2 * NSLOT)],
        ],
    )(z, src, dst)


def kernel(z, edge_index):
    src = edge_index[0].astype(jnp.int32)
    dst = edge_index[1].astype(jnp.int32)
    return _gae_decode(z, src, dst)


# speedup vs baseline: 1.1538x; 1.0757x over previous
import jax
import jax.numpy as jnp
from jax import lax
from jax.experimental import pallas as pl
from jax.experimental.pallas import tpu as pltpu
from jax.experimental.pallas import tpu_sc as plsc

D = 128
E = 320000
NC, NS, L = 2, 16, 16
NW = NC * NS
EW = E // NW
C = 80
K = EW // C
G = C // L
PS = L + 1
NSLOT = 2


def _body(z_hbm, src_hbm, dst_hbm, out_hbm,
          idx_s, idx_d, rowbufs, p1, out_buf, sems):
    wid = lax.axis_index("c") * NS + lax.axis_index("s")
    base = wid * EW
    iota = lax.iota(jnp.int32, L)
    iota_ps = iota * PS

    slots = tuple(
        (rowbufs[2 * s], rowbufs[2 * s + 1], sems[2 * s], sems[2 * s + 1])
        for s in range(NSLOT)
    )

    pltpu.sync_copy(src_hbm.at[pl.ds(base, EW)], idx_s)
    pltpu.sync_copy(dst_hbm.at[pl.ds(base, EW)], idx_d)

    def issue(c, slot):
        sr, dr, ss, sd = slots[slot]
        pltpu.async_copy(z_hbm.at[idx_s.at[pl.ds(c * C, C)]], sr, ss)
        pltpu.async_copy(z_hbm.at[idx_d.at[pl.ds(c * C, C)]], dr, sd)

    def wait(c, slot):
        sr, dr, ss, sd = slots[slot]
        pltpu.make_async_copy(z_hbm.at[idx_s.at[pl.ds(c * C, C)]], sr, ss).wait()
        pltpu.make_async_copy(z_hbm.at[idx_d.at[pl.ds(c * C, C)]], dr, sd).wait()

    def compute(c, slot):
        sr, dr, _, _ = slots[slot]

        def group_body(g, carry):
            for i in range(L):
                e = g * L + i
                p = None
                for k in range(D // L):
                    vs = sr[e, pl.ds(k * L, L)]
                    vd = dr[e, pl.ds(k * L, L)]
                    prod = vs * vd
                    p = prod if p is None else p + prod
                p1[pl.ds(i * PS, L)] = p
            acc = plsc.load_gather(p1, [iota_ps])
            for j in range(1, L):
                acc = acc + plsc.load_gather(p1, [iota_ps + j])
            out = 1.0 / (1.0 + jnp.exp(-acc))
            out_buf[pl.ds(c * C + g * L, L)] = out
            return carry

        lax.fori_loop(0, G, group_body, 0, unroll=False)

    for s in range(NSLOT):
        issue(s, s)

    def quad_body(q, carry):
        for s in range(NSLOT):
            c = NSLOT * q + s
            wait(c, s)
            compute(c, s)

            @pl.when(c + NSLOT < K)
            def _():
                issue(c + NSLOT, s)

        return carry

    lax.fori_loop(0, K // NSLOT, quad_body, 0, unroll=False)
    c_tail = K // NSLOT * NSLOT
    wait(c_tail, 0)
    compute(c_tail, 0)

    pltpu.sync_copy(out_buf, out_hbm.at[pl.ds(base, EW)])


@jax.jit
def _gae_decode(z, src, dst):
    mesh = plsc.VectorSubcoreMesh(core_axis_name="c", subcore_axis_name="s")
    return pl.kernel(
        _body,
        out_type=jax.ShapeDtypeStruct((E,), jnp.float32),
        mesh=mesh,
        compiler_params=pltpu.CompilerParams(needs_layout_passes=False),
        scratch_types=[
            pltpu.VMEM((EW,), jnp.int32),
            pltpu.VMEM((EW,), jnp.int32),
            [pltpu.VMEM((C, D), jnp.float32) for _ in range(2 * NSLOT)],
            pltpu.VMEM((L * PS,), jnp.float32),
            pltpu.VMEM((EW,), jnp.float32),
            [pltpu.SemaphoreType.DMA for _ in range(2 * NSLOT)],
        ],
    )(z, src, dst)


def kernel(z, edge_index):
    src = edge_index[0].astype(jnp.int32)
    dst = edge_index[1].astype(jnp.int32)
    return _gae_decode(z, src, dst)
